# baseline (speedup 1.0000x reference)
import jax
import jax.numpy as jnp
from jax import lax
from jax.experimental import pallas as pl
from jax.experimental.pallas import tpu as pltpu
from jax.experimental.pallas import tpu_sc as plsc

N_NODES = 10000
D_FEAT = 256
HALF = D_FEAT // 2
NUM_SC = 2
NUM_TEC = 16
CHUNK = 128

ACC_ROWS = 10016
ZERO_ROWS = ACC_ROWS // NUM_TEC
OUT_ROWS = 624
TAIL_ROWS = N_NODES - NUM_TEC * OUT_ROWS


def _sc_kernel(e_pad, n_chunks):
    per_tile = n_chunks * CHUNK
    n_pairs = n_chunks // 2
    assert n_pairs % 2 == 0

    def body(xs_hbm, src_hbm, dst_hbm, zer_hbm, out_hbm,
             src00, src01, src10, src11, dst00, dst01, dst10, dst11,
             rows0, rows1, acc, zsem, gsem0, gsem1, ssem0, ssem1):
        src_v = ((src00, src01), (src10, src11))
        dst_v = ((dst00, dst01), (dst10, dst11))
        rows = (rows0, rows1)
        gsem = (gsem0, gsem1)
        ssem = (ssem0, ssem1)
        c = lax.axis_index("c")
        s = lax.axis_index("s")
        base = c * e_pad + s * per_tile

        pltpu.async_copy(
            zer_hbm, acc.at[pl.ds(s * ZERO_ROWS, ZERO_ROWS)], zsem).wait()
        plsc.subcore_barrier()

        def iload(p, a):
            e0 = base + p * (2 * CHUNK)
            pltpu.sync_copy(src_hbm.at[pl.ds(e0, CHUNK)], src_v[a][0])
            pltpu.sync_copy(src_hbm.at[pl.ds(e0 + CHUNK, CHUNK)], src_v[a][1])
            pltpu.sync_copy(dst_hbm.at[pl.ds(e0, CHUNK)], dst_v[a][0])
            pltpu.sync_copy(dst_hbm.at[pl.ds(e0 + CHUNK, CHUNK)], dst_v[a][1])

        def gather_start(a, i):
            pltpu.async_copy(xs_hbm.at[src_v[a][i]], rows[i], gsem[i])

        def gather_wait(a, i):
            pltpu.make_async_copy(xs_hbm.at[src_v[a][i]], rows[i],
                                  gsem[i]).wait()

        def scatter_start(a, i):
            pltpu.async_copy(rows[i], acc.at[dst_v[a][i]], ssem[i], add=True)

        def scatter_wait(a, i):
            pltpu.make_async_copy(rows[i], acc.at[dst_v[a][i]],
                                  ssem[i]).wait()

        iload(0, 0)

        def outer(o, carry):
            for a in range(2):
                p = 2 * o + a
                gather_start(a, 0)
                gather_start(a, 1)
                iload(p + 1, 1 - a)
                gather_wait(a, 0)
                gather_wait(a, 1)
                scatter_start(a, 0)
                scatter_wait(a, 0)
                scatter_start(a, 1)
                scatter_wait(a, 1)
            return carry

        lax.fori_loop(0, n_pairs // 2, outer, 0)
        plsc.subcore_barrier()

        pltpu.sync_copy(
            acc.at[pl.ds(s * OUT_ROWS, OUT_ROWS)],
            out_hbm.at[pl.ds(s * OUT_ROWS, OUT_ROWS), pl.ds(c * HALF, HALF)])

        @pl.when(s == NUM_TEC - 1)
        def _tail():
            r0 = NUM_TEC * OUT_ROWS
            pltpu.sync_copy(
                acc.at[pl.ds(r0, TAIL_ROWS)],
                out_hbm.at[pl.ds(r0, TAIL_ROWS), pl.ds(c * HALF, HALF)])

    mesh = plsc.VectorSubcoreMesh(core_axis_name="c", subcore_axis_name="s")
    return pl.kernel(
        body,
        out_type=jax.ShapeDtypeStruct((N_NODES, D_FEAT), jnp.float32),
        mesh=mesh,
        scratch_types=(
            [pltpu.VMEM((CHUNK,), jnp.int32)] * 8
            + [pltpu.VMEM((CHUNK, HALF), jnp.float32)] * 2
            + [pltpu.VMEM_SHARED((ACC_ROWS, HALF), jnp.float32)]
            + [pltpu.SemaphoreType.DMA] * 5
        ),
    )


@jax.jit
def kernel(x, up_index, down_index):
    n_edges = up_index.shape[1] + down_index.shape[1]
    align = NUM_TEC * CHUNK * 4
    e_pad = ((n_edges + align - 1) // align) * align
    n_chunks = e_pad // (NUM_TEC * CHUNK)
    pad = e_pad - n_edges

    src = jnp.concatenate(
        [up_index[0], down_index[0], jnp.zeros((pad,), up_index.dtype)]
    ).astype(jnp.int32)
    dst = jnp.concatenate(
        [up_index[1], down_index[1],
         jnp.full((pad,), N_NODES, up_index.dtype)]
    ).astype(jnp.int32)
    extra_s = jnp.zeros((2 * CHUNK,), jnp.int32)
    extra_d = jnp.full((2 * CHUNK,), N_NODES, jnp.int32)
    src_all = jnp.concatenate([src, src + N_NODES, extra_s])
    dst_all = jnp.concatenate([dst, dst, extra_d])
    xs = jnp.concatenate([x[:, :HALF], x[:, HALF:]], axis=0)
    zer = jnp.zeros((ZERO_ROWS, HALF), jnp.float32)

    return _sc_kernel(e_pad, n_chunks)(xs, src_all, dst_all, zer)


# speedup vs baseline: 1.1732x; 1.1732x over previous
import jax
import jax.numpy as jnp
from jax import lax
from jax.experimental import pallas as pl
from jax.experimental.pallas import tpu as pltpu
from jax.experimental.pallas import tpu_sc as plsc

N_NODES = 10000
D_FEAT = 256
HALF = D_FEAT // 2
NUM_SC = 2
NUM_TEC = 16
CHUNK = 128

ACC_ROWS = 10016
ZERO_ROWS = ACC_ROWS // NUM_TEC
OUT_ROWS = 624
TAIL_ROWS = N_NODES - NUM_TEC * OUT_ROWS


def _sc_kernel(e_pad, n_chunks):
    per_tile = n_chunks * CHUNK
    assert n_chunks % 2 == 0

    def body(xs_hbm, src_hbm, dst_hbm, zer_hbm, out_hbm,
             src0, src1, dst0, dst1, rows0, rows1, acc,
             zsem, gsem0, gsem1):
        src_v = (src0, src1)
        dst_v = (dst0, dst1)
        rows = (rows0, rows1)
        gsem = (gsem0, gsem1)
        c = lax.axis_index("c")
        s = lax.axis_index("s")
        base = c * e_pad + s * per_tile

        pltpu.async_copy(
            zer_hbm, acc.at[pl.ds(s * ZERO_ROWS, ZERO_ROWS)], zsem).wait()
        plsc.subcore_barrier()

        def iload(k, b):
            e0 = base + k * CHUNK
            pltpu.sync_copy(src_hbm.at[pl.ds(e0, CHUNK)], src_v[b])
            pltpu.sync_copy(dst_hbm.at[pl.ds(e0, CHUNK)], dst_v[b])

        def gather_start(b):
            pltpu.async_copy(xs_hbm.at[src_v[b]], rows[b], gsem[b])

        def gather_wait(b):
            pltpu.make_async_copy(xs_hbm.at[src_v[b]], rows[b],
                                  gsem[b]).wait()

        iload(0, 0)
        gather_start(0)

        def outer(o, carry):
            for i in range(2):
                k = 2 * o + i
                b = i
                b1 = 1 - i
                iload(k + 1, b1)
                gather_wait(b)
                pltpu.sync_copy(rows[b], acc.at[dst_v[b]], add=True)
                gather_start(b1)
            return carry

        lax.fori_loop(0, n_chunks // 2, outer, 0)
        gather_wait(0)
        plsc.subcore_barrier()

        pltpu.sync_copy(
            acc.at[pl.ds(s * OUT_ROWS, OUT_ROWS)],
            out_hbm.at[pl.ds(s * OUT_ROWS, OUT_ROWS), pl.ds(c * HALF, HALF)])

        @pl.when(s == NUM_TEC - 1)
        def _tail():
            r0 = NUM_TEC * OUT_ROWS
            pltpu.sync_copy(
                acc.at[pl.ds(r0, TAIL_ROWS)],
                out_hbm.at[pl.ds(r0, TAIL_ROWS), pl.ds(c * HALF, HALF)])

    mesh = plsc.VectorSubcoreMesh(core_axis_name="c", subcore_axis_name="s")
    return pl.kernel(
        body,
        out_type=jax.ShapeDtypeStruct((N_NODES, D_FEAT), jnp.float32),
        mesh=mesh,
        scratch_types=(
            [pltpu.VMEM((CHUNK,), jnp.int32)] * 4
            + [pltpu.VMEM((CHUNK, HALF), jnp.float32)] * 2
            + [pltpu.VMEM_SHARED((ACC_ROWS, HALF), jnp.float32)]
            + [pltpu.SemaphoreType.DMA] * 3
        ),
    )


@jax.jit
def kernel(x, up_index, down_index):
    n_edges = up_index.shape[1] + down_index.shape[1]
    align = NUM_TEC * CHUNK * 2
    e_pad = ((n_edges + align - 1) // align) * align
    n_chunks = e_pad // (NUM_TEC * CHUNK)
    pad = e_pad - n_edges

    src = jnp.concatenate(
        [up_index[0], down_index[0], jnp.zeros((pad,), up_index.dtype)]
    ).astype(jnp.int32)
    dst = jnp.concatenate(
        [up_index[1], down_index[1],
         jnp.full((pad,), N_NODES, up_index.dtype)]
    ).astype(jnp.int32)
    extra_s = jnp.zeros((CHUNK,), jnp.int32)
    extra_d = jnp.full((CHUNK,), N_NODES, jnp.int32)
    src_all = jnp.concatenate([src, src + N_NODES, extra_s])
    dst_all = jnp.concatenate([dst, dst, extra_d])
    xs = jnp.concatenate([x[:, :HALF], x[:, HALF:]], axis=0)
    zer = jnp.zeros((ZERO_ROWS, HALF), jnp.float32)

    return _sc_kernel(e_pad, n_chunks)(xs, src_all, dst_all, zer)
